# Initial kernel scaffold; baseline (speedup 1.0000x reference)
import jax
import jax.numpy as jnp
from jax.experimental import pallas as pl


def kernel(hidden_states, gate_w, w_gate, w_up, w_down):
    raise NotImplementedError("write your pallas kernel here")


# speedup vs baseline: 1.5900x; 1.5900x over previous
import functools

import jax
import jax.numpy as jnp
from jax.experimental import pallas as pl
from jax.experimental.pallas import tpu as pltpu

T = 2048
D = 1024
F = 768
E = 8
TOP_K = 2

TM = 256


def _routing_weights(x, gate_w):
    logits = jax.lax.dot_general(
        x, gate_w, (((1,), (1,)), ((), ())),
        preferred_element_type=jnp.float32)
    iota = jax.lax.broadcasted_iota(jnp.int32, logits.shape, 1)
    v1 = jnp.max(logits, axis=1, keepdims=True)
    i1 = jnp.min(jnp.where(logits == v1, iota, E), axis=1, keepdims=True)
    hot1 = iota == i1
    neg = jnp.float32(-1e30)
    l2 = jnp.where(hot1, neg, logits)
    v2 = jnp.max(l2, axis=1, keepdims=True)
    i2 = jnp.min(jnp.where(l2 == v2, iota, E), axis=1, keepdims=True)
    hot2 = iota == i2
    w1 = jax.nn.sigmoid(v1 - v2)
    w2 = 1.0 - w1
    return jnp.where(hot1, w1, 0.0) + jnp.where(hot2, w2, 0.0)


def _moe_body(x_ref, gate_ref, wg_ref, wu_ref, wd_ref, out_ref, acc, rout):
    e = pl.program_id(0)
    t = pl.program_id(1)

    x = x_ref[...]
    rows = pl.ds(t * TM, TM)

    @pl.when(e == 0)
    def _():
        rout[rows, :] = _routing_weights(x, gate_ref[...])
        acc[rows, :] = jnp.zeros((TM, D), jnp.float32)

    g = jax.lax.dot_general(x, wg_ref[0], (((1,), (1,)), ((), ())),
                            preferred_element_type=jnp.float32)
    u = jax.lax.dot_general(x, wu_ref[0], (((1,), (1,)), ((), ())),
                            preferred_element_type=jnp.float32)
    h = (g * jax.nn.sigmoid(g)) * u
    y = jax.lax.dot_general(h, wd_ref[0], (((1,), (1,)), ((), ())),
                            preferred_element_type=jnp.float32)
    r_blk = rout[rows, :]
    lane = jax.lax.broadcasted_iota(jnp.int32, r_blk.shape, 1)
    w_e = jnp.sum(jnp.where(lane == e, r_blk, 0.0), axis=1, keepdims=True)
    acc[rows, :] += w_e * y

    @pl.when(e == E - 1)
    def _flush():
        out_ref[...] = acc[rows, :]


def kernel(hidden_states, gate_w, w_gate, w_up, w_down):
    orig_shape = hidden_states.shape
    x = hidden_states.reshape(-1, orig_shape[-1])
    out = pl.pallas_call(
        _moe_body,
        grid=(E, T // TM),
        in_specs=[
            pl.BlockSpec((TM, D), lambda e, t: (t, 0)),
            pl.BlockSpec((E, D), lambda e, t: (0, 0)),
            pl.BlockSpec((1, F, D), lambda e, t: (e, 0, 0)),
            pl.BlockSpec((1, F, D), lambda e, t: (e, 0, 0)),
            pl.BlockSpec((1, D, F), lambda e, t: (e, 0, 0)),
        ],
        out_specs=pl.BlockSpec((TM, D), lambda e, t: (t, 0)),
        out_shape=jax.ShapeDtypeStruct((T, D), jnp.float32),
        scratch_shapes=[
            pltpu.VMEM((T, D), jnp.float32),
            pltpu.VMEM((T, E), jnp.float32),
        ],
    )(x, gate_w, w_gate, w_up, w_down)
    return out.reshape(orig_shape)
